# baseline (speedup 1.0000x reference)
import functools

import jax
import jax.numpy as jnp
import numpy as np
from jax import lax
from jax.experimental import pallas as pl
from jax.experimental.pallas import tpu as pltpu
from jax.experimental.pallas import tpu_sc as plsc

_N = 10000
_D = 128
_E = 320000
_L = 4
_ALPHA = 0.1
_THETA = 0.5

_NC = 2
_NS = 16
_CHUNK = 64
_NISS = -(-_E // (_NC * _NS * _CHUNK))
_NISS += _NISS % 2
_EPAD = _NC * _NS * _NISS * _CHUNK
_RPT = 632
_NPAD = _RPT * _NS

_ROWBLK = 1000


def _seg_sum_sc(h, src4, dst4, zeros):
    mesh = plsc.VectorSubcoreMesh(
        core_axis_name="c", subcore_axis_name="s",
        num_cores=_NC, num_subcores=_NS)

    @functools.partial(
        pl.kernel,
        out_type=jax.ShapeDtypeStruct((_NC, _NPAD, _D), jnp.float32),
        mesh=mesh,
        scratch_types=[
            pltpu.VMEM((_NISS * _CHUNK,), jnp.int32),
            pltpu.VMEM((_NISS * _CHUNK,), jnp.int32),
            pltpu.VMEM((_CHUNK, _D), jnp.float32),
            pltpu.VMEM((_CHUNK, _D), jnp.float32),
            pltpu.VMEM_SHARED((_NPAD, _D), jnp.float32),
            pltpu.SemaphoreType.DMA,
            pltpu.SemaphoreType.DMA,
        ],
    )
    def k(h_hbm, src_hbm, dst_hbm, z_hbm, out_hbm, src_v, dst_v, rows0,
          rows1, agg_sh, sem0, sem1):
        cid = lax.axis_index("c")
        sid = lax.axis_index("s")
        pltpu.sync_copy(z_hbm, agg_sh.at[pl.ds(sid * _RPT, _RPT)])
        pltpu.sync_copy(src_hbm.at[cid, sid], src_v)
        pltpu.sync_copy(dst_hbm.at[cid, sid], dst_v)
        plsc.subcore_barrier()

        def body(i, carry):
            j = 2 * i
            ia = pl.ds(j * _CHUNK, _CHUNK)
            ib = pl.ds((j + 1) * _CHUNK, _CHUNK)
            d0 = pltpu.async_copy(h_hbm.at[src_v.at[ia]], rows0, sem0)
            d1 = pltpu.async_copy(h_hbm.at[src_v.at[ib]], rows1, sem1)
            d0.wait()
            d1.wait()
            pltpu.sync_copy(rows0, agg_sh.at[dst_v.at[ia]], add=True)
            pltpu.sync_copy(rows1, agg_sh.at[dst_v.at[ib]], add=True)
            return carry

        lax.fori_loop(0, _NISS // 2, body, 0)
        plsc.subcore_barrier()
        pltpu.sync_copy(agg_sh.at[pl.ds(sid * _RPT, _RPT)],
                        out_hbm.at[cid, pl.ds(sid * _RPT, _RPT)])

    return k(h, src4, dst4, zeros)


def _init_body(x_ref, w_ref, b_ref, o_ref):
    o_ref[...] = jnp.maximum(
        jnp.dot(x_ref[...], w_ref[...], preferred_element_type=jnp.float32)
        + b_ref[...], 0.0)


def _dense_init(x, w_t, b2):
    return pl.pallas_call(
        _init_body,
        grid=(_N // _ROWBLK,),
        in_specs=[
            pl.BlockSpec((_ROWBLK, _D), lambda i: (i, 0)),
            pl.BlockSpec((_D, _D), lambda i: (0, 0)),
            pl.BlockSpec((1, _D), lambda i: (0, 0)),
        ],
        out_specs=pl.BlockSpec((_ROWBLK, _D), lambda i: (i, 0)),
        out_shape=jax.ShapeDtypeStruct((_N, _D), jnp.float32),
    )(x, w_t, b2)


def _combine(partials, x0, w, beta):
    def body(p_ref, q_ref, x0_ref, w_ref, o_ref):
        hmix = (1.0 - _ALPHA) * (p_ref[0] + q_ref[0]) + _ALPHA * x0_ref[...]
        o_ref[...] = jnp.maximum(
            (1.0 - beta) * hmix
            + beta * jnp.dot(hmix, w_ref[...],
                             preferred_element_type=jnp.float32), 0.0)

    return pl.pallas_call(
        body,
        grid=(_N // _ROWBLK,),
        in_specs=[
            pl.BlockSpec((1, _ROWBLK, _D), lambda i: (0, i, 0)),
            pl.BlockSpec((1, _ROWBLK, _D), lambda i: (1, i, 0)),
            pl.BlockSpec((_ROWBLK, _D), lambda i: (i, 0)),
            pl.BlockSpec((_D, _D), lambda i: (0, 0)),
        ],
        out_specs=pl.BlockSpec((_ROWBLK, _D), lambda i: (i, 0)),
        out_shape=jax.ShapeDtypeStruct((_N, _D), jnp.float32),
    )(partials, partials, x0, w)


def kernel(x, adj_t, lin_W, lin_b, convW):
    dst = adj_t[0]
    src = adj_t[1]
    pad = _EPAD - _E
    src4 = jnp.concatenate(
        [src, jnp.zeros((pad,), jnp.int32)]).reshape(_NC, _NS, _NISS * _CHUNK)
    dst4 = jnp.concatenate(
        [dst, jnp.full((pad,), _N, jnp.int32)]).reshape(
            _NC, _NS, _NISS * _CHUNK)
    zeros = jnp.zeros((_RPT, _D), jnp.float32)

    h = _dense_init(x, lin_W.T, lin_b.reshape(1, _D))
    x0 = h
    for l in range(_L):
        beta = float(np.log(_THETA / (l + 1) + 1.0))
        partials = _seg_sum_sc(h, src4, dst4, zeros)
        h = _combine(partials, x0, convW[l], beta)
    return h


# speedup vs baseline: 1.1816x; 1.1816x over previous
import functools

import jax
import jax.numpy as jnp
import numpy as np
from jax import lax
from jax.experimental import pallas as pl
from jax.experimental.pallas import tpu as pltpu
from jax.experimental.pallas import tpu_sc as plsc

_N = 10000
_D = 128
_E = 320000
_L = 4
_ALPHA = 0.1
_THETA = 0.5

_NC = 2
_NS = 16
_CHUNK = 64
_NISS = -(-_E // (_NC * _NS * _CHUNK))
_EPAD = _NC * _NS * _NISS * _CHUNK
_RPT = 632
_NPAD = _RPT * _NS

_ROWBLK = 1000


def _seg_sum_sc(h, src4, dst4, zeros):
    mesh = plsc.VectorSubcoreMesh(
        core_axis_name="c", subcore_axis_name="s",
        num_cores=_NC, num_subcores=_NS)

    @functools.partial(
        pl.kernel,
        out_type=jax.ShapeDtypeStruct((_NC, _NPAD, _D), jnp.float32),
        mesh=mesh,
        scratch_types=[
            pltpu.VMEM((_NISS, _CHUNK), jnp.int32),
            pltpu.VMEM((_NISS, _CHUNK), jnp.int32),
            pltpu.VMEM((_CHUNK, _D), jnp.float32),
            pltpu.VMEM_SHARED((_NPAD, _D), jnp.float32),
            pltpu.SemaphoreType.DMA,
        ],
    )
    def k(h_hbm, src_hbm, dst_hbm, z_hbm, out_hbm, src_v, dst_v, rows_v,
          agg_sh, sem):
        cid = lax.axis_index("c")
        sid = lax.axis_index("s")
        pltpu.sync_copy(z_hbm, agg_sh.at[pl.ds(sid * _RPT, _RPT)])
        pltpu.sync_copy(src_hbm.at[cid, sid], src_v)
        pltpu.sync_copy(dst_hbm.at[cid, sid], dst_v)
        plsc.subcore_barrier()

        def body(j, carry):
            pltpu.async_copy(h_hbm.at[src_v.at[j]], rows_v, sem).wait()
            pltpu.sync_copy(rows_v, agg_sh.at[dst_v.at[j]], add=True)
            return carry

        lax.fori_loop(0, _NISS, body, 0)
        plsc.subcore_barrier()
        pltpu.sync_copy(agg_sh.at[pl.ds(sid * _RPT, _RPT)],
                        out_hbm.at[cid, pl.ds(sid * _RPT, _RPT)])

    return k(h, src4, dst4, zeros)


def _init_body(x_ref, w_ref, b_ref, o_ref):
    o_ref[...] = jnp.maximum(
        jnp.dot(x_ref[...], w_ref[...], preferred_element_type=jnp.float32)
        + b_ref[...], 0.0)


def _dense_init(x, w_t, b2):
    return pl.pallas_call(
        _init_body,
        grid=(_N // _ROWBLK,),
        in_specs=[
            pl.BlockSpec((_ROWBLK, _D), lambda i: (i, 0)),
            pl.BlockSpec((_D, _D), lambda i: (0, 0)),
            pl.BlockSpec((1, _D), lambda i: (0, 0)),
        ],
        out_specs=pl.BlockSpec((_ROWBLK, _D), lambda i: (i, 0)),
        out_shape=jax.ShapeDtypeStruct((_N, _D), jnp.float32),
    )(x, w_t, b2)


def _combine(partials, x0, w, beta):
    def body(p_ref, q_ref, x0_ref, w_ref, o_ref):
        hmix = (1.0 - _ALPHA) * (p_ref[0] + q_ref[0]) + _ALPHA * x0_ref[...]
        o_ref[...] = jnp.maximum(
            (1.0 - beta) * hmix
            + beta * jnp.dot(hmix, w_ref[...],
                             preferred_element_type=jnp.float32), 0.0)

    return pl.pallas_call(
        body,
        grid=(_N // _ROWBLK,),
        in_specs=[
            pl.BlockSpec((1, _ROWBLK, _D), lambda i: (0, i, 0)),
            pl.BlockSpec((1, _ROWBLK, _D), lambda i: (1, i, 0)),
            pl.BlockSpec((_ROWBLK, _D), lambda i: (i, 0)),
            pl.BlockSpec((_D, _D), lambda i: (0, 0)),
        ],
        out_specs=pl.BlockSpec((_ROWBLK, _D), lambda i: (i, 0)),
        out_shape=jax.ShapeDtypeStruct((_N, _D), jnp.float32),
    )(partials, partials, x0, w)


def kernel(x, adj_t, lin_W, lin_b, convW):
    dst = adj_t[0]
    src = adj_t[1]
    pad = _EPAD - _E
    src4 = jnp.concatenate(
        [src, jnp.zeros((pad,), jnp.int32)]).reshape(_NC, _NS, _NISS, _CHUNK)
    dst4 = jnp.concatenate(
        [dst, jnp.full((pad,), _N, jnp.int32)]).reshape(_NC, _NS, _NISS, _CHUNK)
    zeros = jnp.zeros((_RPT, _D), jnp.float32)

    h = _dense_init(x, lin_W.T, lin_b.reshape(1, _D))
    x0 = h
    for l in range(_L):
        beta = float(np.log(_THETA / (l + 1) + 1.0))
        partials = _seg_sum_sc(h, src4, dst4, zeros)
        h = _combine(partials, x0, convW[l], beta)
    return h
